# baseline (speedup 1.0000x reference)
import functools

import jax
import jax.numpy as jnp
from jax import lax
from jax.experimental import pallas as pl
from jax.experimental.pallas import tpu as pltpu
from jax.experimental.pallas import tpu_sc as plsc

N = 10000
E = 320000
DH = 128
DO = 40
NUM_ITER = 4

NC = 2
NS = 16
NW = NC * NS
CHUNK = 128
NCH = 80
EPT = NCH * CHUNK
E_PAD = NW * EPT
NPAD = 10112
RPT = NPAD // NS

_mesh = plsc.VectorSubcoreMesh(
    core_axis_name="c", subcore_axis_name="s", num_cores=NC, num_subcores=NS)

NPAD_D = 10240
RPT_D = NPAD_D // NS
WAVE = 8


@functools.partial(
    pl.kernel,
    out_type=jax.ShapeDtypeStruct((2 * NPAD_D,), jnp.float32),
    mesh=_mesh,
    scratch_types=[
        pltpu.VMEM((NCH, CHUNK), jnp.int32),
        pltpu.VMEM((CHUNK,), jnp.float32),
        pltpu.VMEM((RPT_D,), jnp.float32),
        pltpu.VMEM_SHARED((NPAD_D,), jnp.float32),
        pltpu.SemaphoreType.DMA,
    ],
)
def _deg(dst_hbm, out_hbm, dall, ones_v, zbuf, dacc, ss):
    c = lax.axis_index("c")
    s = lax.axis_index("s")
    wid = s * NC + c
    pltpu.sync_copy(dst_hbm.at[pl.ds(wid * NCH, NCH)], dall)

    def fill(j, _):
        ones_v[pl.ds(j * 16, 16)] = jnp.ones((16,), jnp.float32)
        zbuf[pl.ds(j * 16, 16)] = jnp.zeros((16,), jnp.float32)
        return 0

    lax.fori_loop(0, CHUNK // 16, fill, 0)

    def fillz(j, _):
        zbuf[pl.ds(j * 16, 16)] = jnp.zeros((16,), jnp.float32)
        return 0

    lax.fori_loop(CHUNK // 16, RPT_D // 16, fillz, 0)
    r0 = s * RPT_D
    pltpu.sync_copy(zbuf, dacc.at[pl.ds(r0, RPT_D)])
    plsc.subcore_barrier()

    def wave(w, _):
        for j in range(WAVE):
            pltpu.async_copy(ones_v, dacc.at[dall.at[w * WAVE + j]], ss,
                             add=True)
        for j in range(WAVE):
            pltpu.make_async_copy(ones_v, dacc.at[dall.at[w * WAVE + j]],
                                  ss).wait()
        return 0

    lax.fori_loop(0, NCH // WAVE, wave, 0)
    plsc.subcore_barrier()
    pltpu.sync_copy(dacc.at[pl.ds(r0, RPT_D)], zbuf)
    pltpu.sync_copy(zbuf, out_hbm.at[pl.ds(c * NPAD_D + r0, RPT_D)])


@functools.partial(
    pl.kernel,
    out_type=jax.ShapeDtypeStruct((2 * NPAD, DH), jnp.float32),
    mesh=_mesh,
    scratch_types=[
        pltpu.VMEM((NCH, CHUNK), jnp.int32),
        pltpu.VMEM((CHUNK,), jnp.int32),
        pltpu.VMEM((CHUNK,), jnp.int32),
        pltpu.VMEM((CHUNK, DH), jnp.float32),
        pltpu.VMEM((CHUNK, DH), jnp.float32),
        pltpu.VMEM_SHARED((NPAD, DH), jnp.float32),
        pltpu.SemaphoreType.DMA,
        pltpu.SemaphoreType.DMA,
        pltpu.SemaphoreType.DMA,
        pltpu.SemaphoreType.DMA,
        pltpu.SemaphoreType.DMA,
        pltpu.SemaphoreType.DMA,
    ],
)
def _agg(y_hbm, src_hbm, dst_hbm, zer_hbm, out_hbm,
         sall, didx0, didx1, rb0, rb1, acc, gs0, gs1, ds0, ds1, ss0, ss1):
    c = lax.axis_index("c")
    s = lax.axis_index("s")
    wid = s * NC + c
    pltpu.sync_copy(src_hbm.at[pl.ds(wid * NCH, NCH)], sall)
    pltpu.sync_copy(zer_hbm, rb0)
    r0 = s * RPT
    for t in range(RPT // CHUNK):
        pltpu.sync_copy(rb0, acc.at[pl.ds(r0 + t * CHUNK, CHUNK)])
    rem = RPT - (RPT // CHUNK) * CHUNK
    pltpu.sync_copy(rb0.at[pl.ds(0, rem)],
                    acc.at[pl.ds(r0 + (RPT // CHUNK) * CHUNK, rem)])
    pltpu.async_copy(dst_hbm.at[wid * NCH], didx0, ds0)
    pltpu.async_copy(dst_hbm.at[wid * NCH + 1], didx1, ds1)
    pltpu.async_copy(y_hbm.at[sall.at[0]], rb0, gs0)
    pltpu.async_copy(y_hbm.at[sall.at[1]], rb1, gs1)
    plsc.subcore_barrier()

    def body(i, _):
        k = 2 * i

        pltpu.make_async_copy(y_hbm.at[sall.at[k]], rb0, gs0).wait()
        pltpu.make_async_copy(dst_hbm.at[wid * NCH + k], didx0, ds0).wait()
        pltpu.async_copy(rb0, acc.at[didx0], ss0, add=True)

        pltpu.make_async_copy(y_hbm.at[sall.at[k + 1]], rb1, gs1).wait()
        pltpu.make_async_copy(dst_hbm.at[wid * NCH + k + 1], didx1, ds1).wait()
        pltpu.async_copy(rb1, acc.at[didx1], ss1, add=True)

        @pl.when(k + 2 < NCH)
        def _():
            pltpu.make_async_copy(rb0, acc.at[didx0], ss0).wait()
            pltpu.async_copy(dst_hbm.at[wid * NCH + k + 2], didx0, ds0)
            pltpu.async_copy(y_hbm.at[sall.at[k + 2]], rb0, gs0)

        @pl.when(k + 3 < NCH)
        def _():
            pltpu.make_async_copy(rb1, acc.at[didx1], ss1).wait()
            pltpu.async_copy(dst_hbm.at[wid * NCH + k + 3], didx1, ds1)
            pltpu.async_copy(y_hbm.at[sall.at[k + 3]], rb1, gs1)

        return 0

    lax.fori_loop(0, NCH // 2, body, 0)
    pltpu.make_async_copy(rb0, acc.at[didx0], ss0).wait()
    pltpu.make_async_copy(rb1, acc.at[didx1], ss1).wait()
    plsc.subcore_barrier()
    o0 = c * NPAD + r0
    pltpu.sync_copy(acc.at[pl.ds(r0, RPT)], out_hbm.at[pl.ds(o0, RPT)])


BROW = 2000
GRID = N // BROW

_row = lambda i: (i, 0)
_cst = lambda i: (0, 0)
_p0 = lambda i: (0, i, 0)
_p1 = lambda i: (1, i, 0)


def _mm(a, b):
    return jax.lax.dot(a, b, precision=jax.lax.Precision.HIGHEST,
                       preferred_element_type=jnp.float32)


def _enc_body(x_ref, we_ref, be_ref, wg_ref, d0_ref, d1_ref,
              h_ref, y_ref, dinv_ref):
    deg = d0_ref[0] + d1_ref[0] + 1.0
    dinv = lax.rsqrt(deg)
    h = jnp.maximum(_mm(x_ref[...], we_ref[...]) + be_ref[...], 0.0)
    h_ref[...] = h
    dinv_ref[...] = dinv
    y_ref[...] = _mm(h, wg_ref[...]) * dinv


_enc = pl.pallas_call(
    _enc_body,
    grid=(GRID,),
    in_specs=[
        pl.BlockSpec((BROW, DH), _row),
        pl.BlockSpec((DH, DH), _cst),
        pl.BlockSpec((1, DH), _cst),
        pl.BlockSpec((DH, DH), _cst),
        pl.BlockSpec((1, BROW, 1), _p0),
        pl.BlockSpec((1, BROW, 1), _p1),
    ],
    out_specs=(
        pl.BlockSpec((BROW, DH), _row),
        pl.BlockSpec((BROW, DH), _row),
        pl.BlockSpec((BROW, 1), _row),
    ),
    out_shape=(
        jax.ShapeDtypeStruct((N, DH), jnp.float32),
        jax.ShapeDtypeStruct((N, DH), jnp.float32),
        jax.ShapeDtypeStruct((N, 1), jnp.float32),
    ),
)


def _upd_body(h_ref, y_ref, dinv_ref, a0_ref, a1_ref, bg_ref, wg_ref,
              h2_ref, y2_ref):
    agg = a0_ref[0] + a1_ref[0]
    dinv = dinv_ref[...]
    nh = jnp.maximum(dinv * (agg + y_ref[...]) + bg_ref[...], 0.0)
    h2 = 0.5 * h_ref[...] + 0.5 * nh
    h2_ref[...] = h2
    y2_ref[...] = _mm(h2, wg_ref[...]) * dinv


_upd = pl.pallas_call(
    _upd_body,
    grid=(GRID,),
    in_specs=[
        pl.BlockSpec((BROW, DH), _row),
        pl.BlockSpec((BROW, DH), _row),
        pl.BlockSpec((BROW, 1), _row),
        pl.BlockSpec((1, BROW, DH), _p0),
        pl.BlockSpec((1, BROW, DH), _p1),
        pl.BlockSpec((1, DH), _cst),
        pl.BlockSpec((DH, DH), _cst),
    ],
    out_specs=(
        pl.BlockSpec((BROW, DH), _row),
        pl.BlockSpec((BROW, DH), _row),
    ),
    out_shape=(
        jax.ShapeDtypeStruct((N, DH), jnp.float32),
        jax.ShapeDtypeStruct((N, DH), jnp.float32),
    ),
)


def _fin_body(h_ref, y_ref, dinv_ref, a0_ref, a1_ref, bg_ref, wd_ref, bd_ref,
              out_ref):
    agg = a0_ref[0] + a1_ref[0]
    dinv = dinv_ref[...]
    nh = jnp.maximum(dinv * (agg + y_ref[...]) + bg_ref[...], 0.0)
    h2 = 0.5 * h_ref[...] + 0.5 * nh
    logits = _mm(h2, wd_ref[...]) + bd_ref[...]
    m = jnp.max(logits, axis=1, keepdims=True)
    lse = jnp.log(jnp.sum(jnp.exp(logits - m), axis=1, keepdims=True)) + m
    out_ref[...] = logits - lse


_fin = pl.pallas_call(
    _fin_body,
    grid=(GRID,),
    in_specs=[
        pl.BlockSpec((BROW, DH), _row),
        pl.BlockSpec((BROW, DH), _row),
        pl.BlockSpec((BROW, 1), _row),
        pl.BlockSpec((1, BROW, DH), _p0),
        pl.BlockSpec((1, BROW, DH), _p1),
        pl.BlockSpec((1, DH), _cst),
        pl.BlockSpec((DH, DO), _cst),
        pl.BlockSpec((1, DO), _cst),
    ],
    out_specs=pl.BlockSpec((BROW, DO), _row),
    out_shape=jax.ShapeDtypeStruct((N, DO), jnp.float32),
)


def kernel(x, edge_index, W_enc, b_enc, W_gc, b_gc, W_dec, b_dec):
    npt = EPT - E // NW
    ti = jnp.arange(NW, dtype=jnp.int32)[:, None]
    pj = jnp.arange(npt, dtype=jnp.int32)[None, :]
    pad_s = (pj * 89 + ti * 997) % N
    pad_d = N + (pj + ti * 7) % (NPAD - N)
    srcp = jnp.concatenate([edge_index[0].reshape(NW, -1), pad_s],
                           axis=1).reshape(NW * NCH, CHUNK)
    dstp = jnp.concatenate([edge_index[1].reshape(NW, -1), pad_d],
                           axis=1).reshape(NW * NCH, CHUNK)
    zer_r = jnp.zeros((CHUNK, DH), jnp.float32)
    be = b_enc.reshape(1, DH)
    bg = b_gc.reshape(1, DH)
    bd = b_dec.reshape(1, DO)

    degf = _deg(dstp).reshape(2, NPAD_D, 1)
    h, y, dinv = _enc(x, W_enc, be, W_gc, degf, degf)
    for _ in range(NUM_ITER - 1):
        aggf = _agg(y, srcp, dstp, zer_r).reshape(2, NPAD, DH)
        h, y = _upd(h, y, dinv, aggf, aggf, bg, W_gc)
    aggf = _agg(y, srcp, dstp, zer_r).reshape(2, NPAD, DH)
    return _fin(h, y, dinv, aggf, aggf, bg, W_dec, bd)


# speedup vs baseline: 1.2630x; 1.2630x over previous
import functools

import jax
import jax.numpy as jnp
from jax import lax
from jax.experimental import pallas as pl
from jax.experimental.pallas import tpu as pltpu
from jax.experimental.pallas import tpu_sc as plsc

N = 10000
E = 320000
DH = 128
DO = 40
NUM_ITER = 4

NC = 2
NS = 16
NW = NC * NS
CHUNK = 128
NCH = 80
EPT = NCH * CHUNK
E_PAD = NW * EPT
NPAD = 10112
RPT = NPAD // NS

_mesh = plsc.VectorSubcoreMesh(
    core_axis_name="c", subcore_axis_name="s", num_cores=NC, num_subcores=NS)

NPAD_D = 10240
RPT_D = NPAD_D // NS
WAVE = 8


@functools.partial(
    pl.kernel,
    out_type=jax.ShapeDtypeStruct((2 * NPAD_D,), jnp.float32),
    mesh=_mesh,
    scratch_types=[
        pltpu.VMEM((NCH, CHUNK), jnp.int32),
        pltpu.VMEM((CHUNK,), jnp.float32),
        pltpu.VMEM((RPT_D,), jnp.float32),
        pltpu.VMEM_SHARED((NPAD_D,), jnp.float32),
        pltpu.SemaphoreType.DMA,
    ],
)
def _deg(dst_hbm, out_hbm, dall, ones_v, zbuf, dacc, ss):
    c = lax.axis_index("c")
    s = lax.axis_index("s")
    wid = s * NC + c
    pltpu.sync_copy(dst_hbm.at[pl.ds(wid * NCH, NCH)], dall)

    def fill(j, _):
        ones_v[pl.ds(j * 16, 16)] = jnp.ones((16,), jnp.float32)
        zbuf[pl.ds(j * 16, 16)] = jnp.zeros((16,), jnp.float32)
        return 0

    lax.fori_loop(0, CHUNK // 16, fill, 0)

    def fillz(j, _):
        zbuf[pl.ds(j * 16, 16)] = jnp.zeros((16,), jnp.float32)
        return 0

    lax.fori_loop(CHUNK // 16, RPT_D // 16, fillz, 0)
    r0 = s * RPT_D
    pltpu.sync_copy(zbuf, dacc.at[pl.ds(r0, RPT_D)])
    plsc.subcore_barrier()

    def wave(w, _):
        for j in range(WAVE):
            pltpu.async_copy(ones_v, dacc.at[dall.at[w * WAVE + j]], ss,
                             add=True)
        for j in range(WAVE):
            pltpu.make_async_copy(ones_v, dacc.at[dall.at[w * WAVE + j]],
                                  ss).wait()
        return 0

    lax.fori_loop(0, NCH // WAVE, wave, 0)
    plsc.subcore_barrier()
    pltpu.sync_copy(dacc.at[pl.ds(r0, RPT_D)], zbuf)
    pltpu.sync_copy(zbuf, out_hbm.at[pl.ds(c * NPAD_D + r0, RPT_D)])


@functools.partial(
    pl.kernel,
    out_type=jax.ShapeDtypeStruct((2 * NPAD, DH), jnp.float32),
    mesh=_mesh,
    scratch_types=[
        pltpu.VMEM((NCH, CHUNK), jnp.int32),
        pltpu.VMEM((CHUNK,), jnp.int32),
        pltpu.VMEM((CHUNK,), jnp.int32),
        pltpu.VMEM((CHUNK, DH), jnp.float32),
        pltpu.VMEM((CHUNK, DH), jnp.float32),
        pltpu.VMEM_SHARED((NPAD, DH), jnp.float32),
        pltpu.SemaphoreType.DMA,
        pltpu.SemaphoreType.DMA,
        pltpu.SemaphoreType.DMA,
        pltpu.SemaphoreType.DMA,
        pltpu.SemaphoreType.DMA,
        pltpu.SemaphoreType.DMA,
    ],
)
def _agg(y_hbm, src_hbm, dst_hbm, zer_hbm, out_hbm,
         sall, didx0, didx1, rb0, rb1, acc, gs0, gs1, ds0, ds1, ss0, ss1):
    c = lax.axis_index("c")
    s = lax.axis_index("s")
    wid = s * NC + c
    pltpu.sync_copy(src_hbm.at[pl.ds(wid * NCH, NCH)], sall)
    pltpu.sync_copy(zer_hbm, rb0)
    r0 = s * RPT
    rem = RPT - (RPT // CHUNK) * CHUNK
    for t in range(RPT // CHUNK):
        pltpu.async_copy(rb0, acc.at[pl.ds(r0 + t * CHUNK, CHUNK)], ss0)
    pltpu.async_copy(rb0.at[pl.ds(0, rem)],
                     acc.at[pl.ds(r0 + (RPT // CHUNK) * CHUNK, rem)], ss1)
    pltpu.async_copy(dst_hbm.at[wid * NCH], didx0, ds0)
    pltpu.async_copy(dst_hbm.at[wid * NCH + 1], didx1, ds1)
    for t in range(RPT // CHUNK):
        pltpu.make_async_copy(rb0, acc.at[pl.ds(r0 + t * CHUNK, CHUNK)],
                              ss0).wait()
    pltpu.make_async_copy(rb0.at[pl.ds(0, rem)],
                          acc.at[pl.ds(r0 + (RPT // CHUNK) * CHUNK, rem)],
                          ss1).wait()
    pltpu.async_copy(y_hbm.at[sall.at[0]], rb0, gs0)
    pltpu.async_copy(y_hbm.at[sall.at[1]], rb1, gs1)
    plsc.subcore_barrier()

    def body(i, _):
        k = 2 * i

        pltpu.make_async_copy(y_hbm.at[sall.at[k]], rb0, gs0).wait()
        pltpu.make_async_copy(dst_hbm.at[wid * NCH + k], didx0, ds0).wait()
        pltpu.sync_copy(rb0, acc.at[didx0], add=True)

        @pl.when(k + 2 < NCH)
        def _():
            pltpu.async_copy(dst_hbm.at[wid * NCH + k + 2], didx0, ds0)
            pltpu.async_copy(y_hbm.at[sall.at[k + 2]], rb0, gs0)

        pltpu.make_async_copy(y_hbm.at[sall.at[k + 1]], rb1, gs1).wait()
        pltpu.make_async_copy(dst_hbm.at[wid * NCH + k + 1], didx1, ds1).wait()
        pltpu.sync_copy(rb1, acc.at[didx1], add=True)

        @pl.when(k + 3 < NCH)
        def _():
            pltpu.async_copy(dst_hbm.at[wid * NCH + k + 3], didx1, ds1)
            pltpu.async_copy(y_hbm.at[sall.at[k + 3]], rb1, gs1)

        return 0

    lax.fori_loop(0, NCH // 2, body, 0)
    plsc.subcore_barrier()
    o0 = c * NPAD + r0
    pltpu.sync_copy(acc.at[pl.ds(r0, RPT)], out_hbm.at[pl.ds(o0, RPT)])


BROW = 2000
GRID = N // BROW

_row = lambda i: (i, 0)
_cst = lambda i: (0, 0)
_p0 = lambda i: (0, i, 0)
_p1 = lambda i: (1, i, 0)


def _mm(a, b):
    return jax.lax.dot(a, b, precision=jax.lax.Precision.HIGHEST,
                       preferred_element_type=jnp.float32)


def _enc_body(x_ref, we_ref, be_ref, wg_ref, d0_ref, d1_ref,
              h_ref, y_ref, dinv_ref):
    deg = d0_ref[0] + d1_ref[0] + 1.0
    dinv = lax.rsqrt(deg)
    h = jnp.maximum(_mm(x_ref[...], we_ref[...]) + be_ref[...], 0.0)
    h_ref[...] = h
    dinv_ref[...] = dinv
    y_ref[...] = _mm(h, wg_ref[...]) * dinv


_enc = pl.pallas_call(
    _enc_body,
    grid=(GRID,),
    in_specs=[
        pl.BlockSpec((BROW, DH), _row),
        pl.BlockSpec((DH, DH), _cst),
        pl.BlockSpec((1, DH), _cst),
        pl.BlockSpec((DH, DH), _cst),
        pl.BlockSpec((1, BROW, 1), _p0),
        pl.BlockSpec((1, BROW, 1), _p1),
    ],
    out_specs=(
        pl.BlockSpec((BROW, DH), _row),
        pl.BlockSpec((BROW, DH), _row),
        pl.BlockSpec((BROW, 1), _row),
    ),
    out_shape=(
        jax.ShapeDtypeStruct((N, DH), jnp.float32),
        jax.ShapeDtypeStruct((N, DH), jnp.float32),
        jax.ShapeDtypeStruct((N, 1), jnp.float32),
    ),
)


def _upd_body(h_ref, y_ref, dinv_ref, a0_ref, a1_ref, bg_ref, wg_ref,
              h2_ref, y2_ref):
    agg = a0_ref[0] + a1_ref[0]
    dinv = dinv_ref[...]
    nh = jnp.maximum(dinv * (agg + y_ref[...]) + bg_ref[...], 0.0)
    h2 = 0.5 * h_ref[...] + 0.5 * nh
    h2_ref[...] = h2
    y2_ref[...] = _mm(h2, wg_ref[...]) * dinv


_upd = pl.pallas_call(
    _upd_body,
    grid=(GRID,),
    in_specs=[
        pl.BlockSpec((BROW, DH), _row),
        pl.BlockSpec((BROW, DH), _row),
        pl.BlockSpec((BROW, 1), _row),
        pl.BlockSpec((1, BROW, DH), _p0),
        pl.BlockSpec((1, BROW, DH), _p1),
        pl.BlockSpec((1, DH), _cst),
        pl.BlockSpec((DH, DH), _cst),
    ],
    out_specs=(
        pl.BlockSpec((BROW, DH), _row),
        pl.BlockSpec((BROW, DH), _row),
    ),
    out_shape=(
        jax.ShapeDtypeStruct((N, DH), jnp.float32),
        jax.ShapeDtypeStruct((N, DH), jnp.float32),
    ),
)


def _fin_body(h_ref, y_ref, dinv_ref, a0_ref, a1_ref, bg_ref, wd_ref, bd_ref,
              out_ref):
    agg = a0_ref[0] + a1_ref[0]
    dinv = dinv_ref[...]
    nh = jnp.maximum(dinv * (agg + y_ref[...]) + bg_ref[...], 0.0)
    h2 = 0.5 * h_ref[...] + 0.5 * nh
    logits = _mm(h2, wd_ref[...]) + bd_ref[...]
    m = jnp.max(logits, axis=1, keepdims=True)
    lse = jnp.log(jnp.sum(jnp.exp(logits - m), axis=1, keepdims=True)) + m
    out_ref[...] = logits - lse


_fin = pl.pallas_call(
    _fin_body,
    grid=(GRID,),
    in_specs=[
        pl.BlockSpec((BROW, DH), _row),
        pl.BlockSpec((BROW, DH), _row),
        pl.BlockSpec((BROW, 1), _row),
        pl.BlockSpec((1, BROW, DH), _p0),
        pl.BlockSpec((1, BROW, DH), _p1),
        pl.BlockSpec((1, DH), _cst),
        pl.BlockSpec((DH, DO), _cst),
        pl.BlockSpec((1, DO), _cst),
    ],
    out_specs=pl.BlockSpec((BROW, DO), _row),
    out_shape=jax.ShapeDtypeStruct((N, DO), jnp.float32),
)


def kernel(x, edge_index, W_enc, b_enc, W_gc, b_gc, W_dec, b_dec):
    npt = EPT - E // NW
    ti = jnp.arange(NW, dtype=jnp.int32)[:, None]
    pj = jnp.arange(npt, dtype=jnp.int32)[None, :]
    pad_s = (pj * 89 + ti * 997) % N
    pad_d = N + (pj + ti * 7) % (NPAD - N)
    srcp = jnp.concatenate([edge_index[0].reshape(NW, -1), pad_s],
                           axis=1).reshape(NW * NCH, CHUNK)
    dstp = jnp.concatenate([edge_index[1].reshape(NW, -1), pad_d],
                           axis=1).reshape(NW * NCH, CHUNK)
    zer_r = jnp.zeros((CHUNK, DH), jnp.float32)
    be = b_enc.reshape(1, DH)
    bg = b_gc.reshape(1, DH)
    bd = b_dec.reshape(1, DO)

    degf = _deg(dstp).reshape(2, NPAD_D, 1)
    h, y, dinv = _enc(x, W_enc, be, W_gc, degf, degf)
    for _ in range(NUM_ITER - 1):
        aggf = _agg(y, srcp, dstp, zer_r).reshape(2, NPAD, DH)
        h, y = _upd(h, y, dinv, aggf, aggf, bg, W_gc)
    aggf = _agg(y, srcp, dstp, zer_r).reshape(2, NPAD, DH)
    return _fin(h, y, dinv, aggf, aggf, bg, W_dec, bd)


# speedup vs baseline: 1.2905x; 1.0217x over previous
import functools

import jax
import jax.numpy as jnp
from jax import lax
from jax.experimental import pallas as pl
from jax.experimental.pallas import tpu as pltpu
from jax.experimental.pallas import tpu_sc as plsc

N = 10000
E = 320000
DH = 128
DO = 40
NUM_ITER = 4

NC = 2
NS = 16
NW = NC * NS
CHUNK = 128
NCH = 81
EPT = NCH * CHUNK
E_PAD = NW * EPT
NPAD = 10112
RPT = NPAD // NS

_mesh = plsc.VectorSubcoreMesh(
    core_axis_name="c", subcore_axis_name="s", num_cores=NC, num_subcores=NS)

NPAD_D = 10240
RPT_D = NPAD_D // NS
WAVE = 9


@functools.partial(
    pl.kernel,
    out_type=jax.ShapeDtypeStruct((2 * NPAD_D,), jnp.float32),
    mesh=_mesh,
    scratch_types=[
        pltpu.VMEM((NCH, CHUNK), jnp.int32),
        pltpu.VMEM((CHUNK,), jnp.float32),
        pltpu.VMEM((RPT_D,), jnp.float32),
        pltpu.VMEM_SHARED((NPAD_D,), jnp.float32),
        pltpu.SemaphoreType.DMA,
    ],
)
def _deg(dst_hbm, out_hbm, dall, ones_v, zbuf, dacc, ss):
    c = lax.axis_index("c")
    s = lax.axis_index("s")
    wid = s * NC + c

    def ldwave(w, _):
        for j in range(WAVE):
            pltpu.async_copy(dst_hbm.at[wid * NCH + w * WAVE + j],
                             dall.at[w * WAVE + j], ss)
        for j in range(WAVE):
            pltpu.make_async_copy(dst_hbm.at[wid * NCH + w * WAVE + j],
                                  dall.at[w * WAVE + j], ss).wait()
        return 0

    lax.fori_loop(0, NCH // WAVE, ldwave, 0)

    def fill(j, _):
        ones_v[pl.ds(j * 16, 16)] = jnp.ones((16,), jnp.float32)
        zbuf[pl.ds(j * 16, 16)] = jnp.zeros((16,), jnp.float32)
        return 0

    lax.fori_loop(0, CHUNK // 16, fill, 0)

    def fillz(j, _):
        zbuf[pl.ds(j * 16, 16)] = jnp.zeros((16,), jnp.float32)
        return 0

    lax.fori_loop(CHUNK // 16, RPT_D // 16, fillz, 0)
    r0 = s * RPT_D
    pltpu.sync_copy(zbuf, dacc.at[pl.ds(r0, RPT_D)])
    plsc.subcore_barrier()

    def wave(w, _):
        for j in range(WAVE):
            pltpu.async_copy(ones_v, dacc.at[dall.at[w * WAVE + j]], ss,
                             add=True)
        for j in range(WAVE):
            pltpu.make_async_copy(ones_v, dacc.at[dall.at[w * WAVE + j]],
                                  ss).wait()
        return 0

    lax.fori_loop(0, NCH // WAVE, wave, 0)
    plsc.subcore_barrier()
    pltpu.sync_copy(dacc.at[pl.ds(r0, RPT_D)], zbuf)
    pltpu.sync_copy(zbuf, out_hbm.at[pl.ds(c * NPAD_D + r0, RPT_D)])


@functools.partial(
    pl.kernel,
    out_type=jax.ShapeDtypeStruct((2 * NPAD, DH), jnp.float32),
    mesh=_mesh,
    scratch_types=[
        pltpu.VMEM((CHUNK,), jnp.int32),
        pltpu.VMEM((CHUNK,), jnp.int32),
        pltpu.VMEM((CHUNK,), jnp.int32),
        pltpu.VMEM((CHUNK,), jnp.int32),
        pltpu.VMEM((CHUNK,), jnp.int32),
        pltpu.VMEM((CHUNK,), jnp.int32),
        pltpu.VMEM((CHUNK, DH), jnp.float32),
        pltpu.VMEM((CHUNK, DH), jnp.float32),
        pltpu.VMEM((CHUNK, DH), jnp.float32),
        pltpu.VMEM_SHARED((NPAD, DH), jnp.float32),
        pltpu.SemaphoreType.DMA,
        pltpu.SemaphoreType.DMA,
        pltpu.SemaphoreType.DMA,
        pltpu.SemaphoreType.DMA,
        pltpu.SemaphoreType.DMA,
        pltpu.SemaphoreType.DMA,
        pltpu.SemaphoreType.DMA,
        pltpu.SemaphoreType.DMA,
        pltpu.SemaphoreType.DMA,
        pltpu.SemaphoreType.DMA,
        pltpu.SemaphoreType.DMA,
        pltpu.SemaphoreType.DMA,
    ],
)
def _agg(y_hbm, src_hbm, dst_hbm, zer_hbm, out_hbm,
         sx0, sx1, sx2, dx0, dx1, dx2, rb0, rb1, rb2, acc,
         gs0, gs1, gs2, ds0, ds1, ds2, ss0, ss1, ss2, is0, is1, is2):
    c = lax.axis_index("c")
    s = lax.axis_index("s")
    wid = s * NC + c
    e0 = wid * NCH
    sx = (sx0, sx1, sx2)
    dx = (dx0, dx1, dx2)
    rb = (rb0, rb1, rb2)
    gs = (gs0, gs1, gs2)
    ds = (ds0, ds1, ds2)
    ss = (ss0, ss1, ss2)
    iss = (is0, is1, is2)

    pltpu.sync_copy(zer_hbm, rb2)
    r0 = s * RPT
    rem = RPT - (RPT // CHUNK) * CHUNK
    for t in range(RPT // CHUNK):
        pltpu.async_copy(rb2, acc.at[pl.ds(r0 + t * CHUNK, CHUNK)], ss0)
    pltpu.async_copy(rb2.at[pl.ds(0, rem)],
                     acc.at[pl.ds(r0 + (RPT // CHUNK) * CHUNK, rem)], ss1)
    pltpu.async_copy(src_hbm.at[e0], sx0, is0)
    pltpu.async_copy(src_hbm.at[e0 + 1], sx1, is1)
    pltpu.async_copy(src_hbm.at[e0 + 2], sx2, is2)
    pltpu.async_copy(dst_hbm.at[e0], dx0, ds0)
    pltpu.async_copy(dst_hbm.at[e0 + 1], dx1, ds1)
    pltpu.async_copy(dst_hbm.at[e0], dx2, ds2)
    for t in range(RPT // CHUNK):
        pltpu.make_async_copy(rb2, acc.at[pl.ds(r0 + t * CHUNK, CHUNK)],
                              ss0).wait()
    pltpu.make_async_copy(rb2.at[pl.ds(0, rem)],
                          acc.at[pl.ds(r0 + (RPT // CHUNK) * CHUNK, rem)],
                          ss1).wait()
    pltpu.make_async_copy(src_hbm.at[e0], sx0, is0).wait()
    pltpu.async_copy(y_hbm.at[sx0], rb0, gs0)
    pltpu.make_async_copy(src_hbm.at[e0 + 1], sx1, is1).wait()
    pltpu.async_copy(y_hbm.at[sx1], rb1, gs1)
    pltpu.make_async_copy(dst_hbm.at[e0], dx2, ds2).wait()
    pltpu.async_copy(rb2, acc.at[dx2], ss2, add=True)
    plsc.subcore_barrier()

    def slot(t, j):
        b = j
        b2 = (j + 2) % 3
        pltpu.make_async_copy(y_hbm.at[sx[b]], rb[b], gs[b]).wait()

        @pl.when(t + 3 < NCH)
        def _():
            pltpu.async_copy(src_hbm.at[e0 + t + 3], sx[b], iss[b])

        pltpu.make_async_copy(dst_hbm.at[e0 + t], dx[b], ds[b]).wait()
        pltpu.async_copy(rb[b], acc.at[dx[b]], ss[b], add=True)
        pltpu.make_async_copy(rb[b2], acc.at[dx[b2]], ss[b2]).wait()

        @pl.when(t + 2 < NCH)
        def _():
            pltpu.async_copy(dst_hbm.at[e0 + t + 2], dx[b2], ds[b2])
            pltpu.make_async_copy(src_hbm.at[e0 + t + 2], sx[b2],
                                  iss[b2]).wait()
            pltpu.async_copy(y_hbm.at[sx[b2]], rb[b2], gs[b2])

    def body(i, _):
        for j in range(3):
            slot(3 * i + j, j)
        return 0

    lax.fori_loop(0, NCH // 3, body, 0)
    bl = (NCH - 1) % 3
    pltpu.make_async_copy(rb[bl], acc.at[dx[bl]], ss[bl]).wait()
    plsc.subcore_barrier()
    o0 = c * NPAD + r0
    pltpu.sync_copy(acc.at[pl.ds(r0, RPT)], out_hbm.at[pl.ds(o0, RPT)])


BROW = 2000
GRID = N // BROW

_row = lambda i: (i, 0)
_cst = lambda i: (0, 0)
_p0 = lambda i: (0, i, 0)
_p1 = lambda i: (1, i, 0)


def _mm(a, b):
    return jax.lax.dot(a, b, precision=jax.lax.Precision.HIGHEST,
                       preferred_element_type=jnp.float32)


def _enc_body(x_ref, we_ref, be_ref, wg_ref, d0_ref, d1_ref,
              h_ref, y_ref, dinv_ref):
    deg = d0_ref[0] + d1_ref[0] + 1.0
    dinv = lax.rsqrt(deg)
    h = jnp.maximum(_mm(x_ref[...], we_ref[...]) + be_ref[...], 0.0)
    h_ref[...] = h
    dinv_ref[...] = dinv
    y_ref[...] = _mm(h, wg_ref[...]) * dinv


_enc = pl.pallas_call(
    _enc_body,
    grid=(GRID,),
    in_specs=[
        pl.BlockSpec((BROW, DH), _row),
        pl.BlockSpec((DH, DH), _cst),
        pl.BlockSpec((1, DH), _cst),
        pl.BlockSpec((DH, DH), _cst),
        pl.BlockSpec((1, BROW, 1), _p0),
        pl.BlockSpec((1, BROW, 1), _p1),
    ],
    out_specs=(
        pl.BlockSpec((BROW, DH), _row),
        pl.BlockSpec((BROW, DH), _row),
        pl.BlockSpec((BROW, 1), _row),
    ),
    out_shape=(
        jax.ShapeDtypeStruct((N, DH), jnp.float32),
        jax.ShapeDtypeStruct((N, DH), jnp.float32),
        jax.ShapeDtypeStruct((N, 1), jnp.float32),
    ),
)


def _upd_body(h_ref, y_ref, dinv_ref, a0_ref, a1_ref, bg_ref, wg_ref,
              h2_ref, y2_ref):
    agg = a0_ref[0] + a1_ref[0]
    dinv = dinv_ref[...]
    nh = jnp.maximum(dinv * (agg + y_ref[...]) + bg_ref[...], 0.0)
    h2 = 0.5 * h_ref[...] + 0.5 * nh
    h2_ref[...] = h2
    y2_ref[...] = _mm(h2, wg_ref[...]) * dinv


_upd = pl.pallas_call(
    _upd_body,
    grid=(GRID,),
    in_specs=[
        pl.BlockSpec((BROW, DH), _row),
        pl.BlockSpec((BROW, DH), _row),
        pl.BlockSpec((BROW, 1), _row),
        pl.BlockSpec((1, BROW, DH), _p0),
        pl.BlockSpec((1, BROW, DH), _p1),
        pl.BlockSpec((1, DH), _cst),
        pl.BlockSpec((DH, DH), _cst),
    ],
    out_specs=(
        pl.BlockSpec((BROW, DH), _row),
        pl.BlockSpec((BROW, DH), _row),
    ),
    out_shape=(
        jax.ShapeDtypeStruct((N, DH), jnp.float32),
        jax.ShapeDtypeStruct((N, DH), jnp.float32),
    ),
)


def _fin_body(h_ref, y_ref, dinv_ref, a0_ref, a1_ref, bg_ref, wd_ref, bd_ref,
              out_ref):
    agg = a0_ref[0] + a1_ref[0]
    dinv = dinv_ref[...]
    nh = jnp.maximum(dinv * (agg + y_ref[...]) + bg_ref[...], 0.0)
    h2 = 0.5 * h_ref[...] + 0.5 * nh
    logits = _mm(h2, wd_ref[...]) + bd_ref[...]
    m = jnp.max(logits, axis=1, keepdims=True)
    lse = jnp.log(jnp.sum(jnp.exp(logits - m), axis=1, keepdims=True)) + m
    out_ref[...] = logits - lse


_fin = pl.pallas_call(
    _fin_body,
    grid=(GRID,),
    in_specs=[
        pl.BlockSpec((BROW, DH), _row),
        pl.BlockSpec((BROW, DH), _row),
        pl.BlockSpec((BROW, 1), _row),
        pl.BlockSpec((1, BROW, DH), _p0),
        pl.BlockSpec((1, BROW, DH), _p1),
        pl.BlockSpec((1, DH), _cst),
        pl.BlockSpec((DH, DO), _cst),
        pl.BlockSpec((1, DO), _cst),
    ],
    out_specs=pl.BlockSpec((BROW, DO), _row),
    out_shape=jax.ShapeDtypeStruct((N, DO), jnp.float32),
)


def kernel(x, edge_index, W_enc, b_enc, W_gc, b_gc, W_dec, b_dec):
    npt = EPT - E // NW
    ti = jnp.arange(NW, dtype=jnp.int32)[:, None]
    pj = jnp.arange(npt, dtype=jnp.int32)[None, :]
    pad_s = (pj * 89 + ti * 997) % N
    pad_d = N + (pj + ti * 7) % (NPAD - N)
    srcp = jnp.concatenate([edge_index[0].reshape(NW, -1), pad_s],
                           axis=1).reshape(NW * NCH, CHUNK)
    dstp = jnp.concatenate([edge_index[1].reshape(NW, -1), pad_d],
                           axis=1).reshape(NW * NCH, CHUNK)
    zer_r = jnp.zeros((CHUNK, DH), jnp.float32)
    be = b_enc.reshape(1, DH)
    bg = b_gc.reshape(1, DH)
    bd = b_dec.reshape(1, DO)

    degf = _deg(dstp).reshape(2, NPAD_D, 1)
    h, y, dinv = _enc(x, W_enc, be, W_gc, degf, degf)
    for _ in range(NUM_ITER - 1):
        aggf = _agg(y, srcp, dstp, zer_r).reshape(2, NPAD, DH)
        h, y = _upd(h, y, dinv, aggf, aggf, bg, W_gc)
    aggf = _agg(y, srcp, dstp, zer_r).reshape(2, NPAD, DH)
    return _fin(h, y, dinv, aggf, aggf, bg, W_dec, bd)


# speedup vs baseline: 1.2927x; 1.0017x over previous
import functools

import jax
import jax.numpy as jnp
from jax import lax
from jax.experimental import pallas as pl
from jax.experimental.pallas import tpu as pltpu
from jax.experimental.pallas import tpu_sc as plsc

N = 10000
E = 320000
DH = 128
DO = 40
NUM_ITER = 4

NC = 2
NS = 16
NW = NC * NS
CHUNK = 128
NCH = 81
EPT = NCH * CHUNK
NPAD = 10112
RPT = NPAD // NS

_mesh = plsc.VectorSubcoreMesh(
    core_axis_name="c", subcore_axis_name="s", num_cores=NC, num_subcores=NS)

NPAD_D = 10240
RPT_D = NPAD_D // NS
WAVE = 9


@functools.partial(
    pl.kernel,
    out_type=jax.ShapeDtypeStruct((2 * NPAD_D,), jnp.float32),
    mesh=_mesh,
    scratch_types=[
        pltpu.VMEM((NCH, CHUNK), jnp.int32),
        pltpu.VMEM((CHUNK,), jnp.float32),
        pltpu.VMEM((RPT_D,), jnp.float32),
        pltpu.VMEM_SHARED((NPAD_D,), jnp.float32),
        pltpu.SemaphoreType.DMA,
    ],
)
def _deg(dst_hbm, out_hbm, dall, ones_v, zbuf, dacc, ss):
    c = lax.axis_index("c")
    s = lax.axis_index("s")
    wid = s * NC + c

    def ldwave(w, _):
        for j in range(WAVE):
            pltpu.async_copy(dst_hbm.at[wid * NCH + w * WAVE + j],
                             dall.at[w * WAVE + j], ss)
        for j in range(WAVE):
            pltpu.make_async_copy(dst_hbm.at[wid * NCH + w * WAVE + j],
                                  dall.at[w * WAVE + j], ss).wait()
        return 0

    lax.fori_loop(0, NCH // WAVE, ldwave, 0)

    def fill(j, _):
        ones_v[pl.ds(j * 16, 16)] = jnp.ones((16,), jnp.float32)
        zbuf[pl.ds(j * 16, 16)] = jnp.zeros((16,), jnp.float32)
        return 0

    lax.fori_loop(0, CHUNK // 16, fill, 0)

    def fillz(j, _):
        zbuf[pl.ds(j * 16, 16)] = jnp.zeros((16,), jnp.float32)
        return 0

    lax.fori_loop(CHUNK // 16, RPT_D // 16, fillz, 0)
    r0 = s * RPT_D
    pltpu.sync_copy(zbuf, dacc.at[pl.ds(r0, RPT_D)])
    plsc.subcore_barrier()

    def wave(w, _):
        for j in range(WAVE):
            pltpu.async_copy(ones_v, dacc.at[dall.at[w * WAVE + j]], ss,
                             add=True)
        for j in range(WAVE):
            pltpu.make_async_copy(ones_v, dacc.at[dall.at[w * WAVE + j]],
                                  ss).wait()
        return 0

    lax.fori_loop(0, NCH // WAVE, wave, 0)
    plsc.subcore_barrier()
    pltpu.sync_copy(dacc.at[pl.ds(r0, RPT_D)], zbuf)
    pltpu.sync_copy(zbuf, out_hbm.at[pl.ds(c * NPAD_D + r0, RPT_D)])


@functools.partial(
    pl.kernel,
    out_type=jax.ShapeDtypeStruct((2 * NPAD, DH), jnp.float32),
    mesh=_mesh,
    scratch_types=[
        pltpu.VMEM((CHUNK,), jnp.int32),
        pltpu.VMEM((CHUNK,), jnp.int32),
        pltpu.VMEM((CHUNK,), jnp.int32),
        pltpu.VMEM((CHUNK,), jnp.int32),
        pltpu.VMEM((CHUNK,), jnp.int32),
        pltpu.VMEM((CHUNK,), jnp.int32),
        pltpu.VMEM((CHUNK, DH), jnp.float32),
        pltpu.VMEM((CHUNK, DH), jnp.float32),
        pltpu.VMEM((CHUNK, DH), jnp.float32),
        pltpu.VMEM_SHARED((NPAD, DH), jnp.float32),
        pltpu.SemaphoreType.DMA,
        pltpu.SemaphoreType.DMA,
        pltpu.SemaphoreType.DMA,
        pltpu.SemaphoreType.DMA,
        pltpu.SemaphoreType.DMA,
        pltpu.SemaphoreType.DMA,
        pltpu.SemaphoreType.DMA,
        pltpu.SemaphoreType.DMA,
        pltpu.SemaphoreType.DMA,
        pltpu.SemaphoreType.DMA,
        pltpu.SemaphoreType.DMA,
        pltpu.SemaphoreType.DMA,
    ],
)
def _agg(y_hbm, src_hbm, dst_hbm, zer_hbm, out_hbm,
         sx0, sx1, sx2, dx0, dx1, dx2, rb0, rb1, rb2, acc,
         gs0, gs1, gs2, ds0, ds1, ds2, ss0, ss1, ss2, is0, is1, is2):
    c = lax.axis_index("c")
    s = lax.axis_index("s")
    wid = s * NC + c
    e0 = wid * NCH
    sx = (sx0, sx1, sx2)
    dx = (dx0, dx1, dx2)
    rb = (rb0, rb1, rb2)
    gs = (gs0, gs1, gs2)
    ds = (ds0, ds1, ds2)
    ss = (ss0, ss1, ss2)
    iss = (is0, is1, is2)

    pltpu.sync_copy(zer_hbm, rb2)
    r0 = s * RPT
    rem = RPT - (RPT // CHUNK) * CHUNK
    for t in range(RPT // CHUNK):
        pltpu.async_copy(rb2, acc.at[pl.ds(r0 + t * CHUNK, CHUNK)], ss0)
    pltpu.async_copy(rb2.at[pl.ds(0, rem)],
                     acc.at[pl.ds(r0 + (RPT // CHUNK) * CHUNK, rem)], ss1)
    pltpu.async_copy(src_hbm.at[e0], sx0, is0)
    pltpu.async_copy(src_hbm.at[e0 + 1], sx1, is1)
    pltpu.async_copy(src_hbm.at[e0 + 2], sx2, is2)
    pltpu.async_copy(dst_hbm.at[e0], dx0, ds0)
    pltpu.async_copy(dst_hbm.at[e0 + 1], dx1, ds1)
    pltpu.async_copy(dst_hbm.at[e0], dx2, ds2)
    for t in range(RPT // CHUNK):
        pltpu.make_async_copy(rb2, acc.at[pl.ds(r0 + t * CHUNK, CHUNK)],
                              ss0).wait()
    pltpu.make_async_copy(rb2.at[pl.ds(0, rem)],
                          acc.at[pl.ds(r0 + (RPT // CHUNK) * CHUNK, rem)],
                          ss1).wait()
    pltpu.make_async_copy(src_hbm.at[e0], sx0, is0).wait()
    pltpu.async_copy(y_hbm.at[sx0], rb0, gs0)
    pltpu.make_async_copy(src_hbm.at[e0 + 1], sx1, is1).wait()
    pltpu.async_copy(y_hbm.at[sx1], rb1, gs1)
    pltpu.make_async_copy(dst_hbm.at[e0], dx2, ds2).wait()
    pltpu.async_copy(rb2, acc.at[dx2], ss2, add=True)
    plsc.subcore_barrier()

    def slot(t, j):
        b = j
        b2 = (j + 2) % 3
        pltpu.make_async_copy(y_hbm.at[sx[b]], rb[b], gs[b]).wait()

        @pl.when(t + 3 < NCH)
        def _():
            pltpu.async_copy(src_hbm.at[e0 + t + 3], sx[b], iss[b])

        pltpu.make_async_copy(dst_hbm.at[e0 + t], dx[b], ds[b]).wait()
        pltpu.async_copy(rb[b], acc.at[dx[b]], ss[b], add=True)
        pltpu.make_async_copy(rb[b2], acc.at[dx[b2]], ss[b2]).wait()

        @pl.when(t + 2 < NCH)
        def _():
            pltpu.async_copy(dst_hbm.at[e0 + t + 2], dx[b2], ds[b2])
            pltpu.make_async_copy(src_hbm.at[e0 + t + 2], sx[b2],
                                  iss[b2]).wait()
            pltpu.async_copy(y_hbm.at[sx[b2]], rb[b2], gs[b2])

    def body(i, _):
        for j in range(3):
            slot(3 * i + j, j)
        return 0

    lax.fori_loop(0, NCH // 3, body, 0)
    bl = (NCH - 1) % 3
    pltpu.make_async_copy(rb[bl], acc.at[dx[bl]], ss[bl]).wait()
    plsc.subcore_barrier()
    o0 = c * NPAD + r0
    pltpu.sync_copy(acc.at[pl.ds(r0, RPT)], out_hbm.at[pl.ds(o0, RPT)])


BROW = 2000
GRID = N // BROW

_row = lambda i: (i, 0)
_cst = lambda i: (0, 0)
_p0 = lambda i: (0, i, 0)
_p1 = lambda i: (1, i, 0)


def _mm(a, b):
    return jax.lax.dot(a, b, precision=jax.lax.Precision.HIGHEST,
                       preferred_element_type=jnp.float32)


def _enc_body(x_ref, we_ref, be_ref, wg_ref, d0_ref, d1_ref,
              h_ref, y_ref, dinv_ref):
    deg = d0_ref[0] + d1_ref[0] + 1.0
    dinv = lax.rsqrt(deg)
    h = jnp.maximum(_mm(x_ref[...], we_ref[...]) + be_ref[...], 0.0)
    h_ref[...] = h
    dinv_ref[...] = dinv
    y_ref[...] = _mm(h, wg_ref[...]) * dinv


_enc = pl.pallas_call(
    _enc_body,
    grid=(GRID,),
    in_specs=[
        pl.BlockSpec((BROW, DH), _row),
        pl.BlockSpec((DH, DH), _cst),
        pl.BlockSpec((1, DH), _cst),
        pl.BlockSpec((DH, DH), _cst),
        pl.BlockSpec((1, BROW, 1), _p0),
        pl.BlockSpec((1, BROW, 1), _p1),
    ],
    out_specs=(
        pl.BlockSpec((BROW, DH), _row),
        pl.BlockSpec((BROW, DH), _row),
        pl.BlockSpec((BROW, 1), _row),
    ),
    out_shape=(
        jax.ShapeDtypeStruct((N, DH), jnp.float32),
        jax.ShapeDtypeStruct((N, DH), jnp.float32),
        jax.ShapeDtypeStruct((N, 1), jnp.float32),
    ),
)


def _upd_body(h_ref, y_ref, dinv_ref, a0_ref, a1_ref, bg_ref, wg_ref,
              h2_ref, y2_ref):
    agg = a0_ref[0] + a1_ref[0]
    dinv = dinv_ref[...]
    nh = jnp.maximum(dinv * (agg + y_ref[...]) + bg_ref[...], 0.0)
    h2 = 0.5 * h_ref[...] + 0.5 * nh
    h2_ref[...] = h2
    y2_ref[...] = _mm(h2, wg_ref[...]) * dinv


_upd = pl.pallas_call(
    _upd_body,
    grid=(GRID,),
    in_specs=[
        pl.BlockSpec((BROW, DH), _row),
        pl.BlockSpec((BROW, DH), _row),
        pl.BlockSpec((BROW, 1), _row),
        pl.BlockSpec((1, BROW, DH), _p0),
        pl.BlockSpec((1, BROW, DH), _p1),
        pl.BlockSpec((1, DH), _cst),
        pl.BlockSpec((DH, DH), _cst),
    ],
    out_specs=(
        pl.BlockSpec((BROW, DH), _row),
        pl.BlockSpec((BROW, DH), _row),
    ),
    out_shape=(
        jax.ShapeDtypeStruct((N, DH), jnp.float32),
        jax.ShapeDtypeStruct((N, DH), jnp.float32),
    ),
)


def _fin_body(h_ref, y_ref, dinv_ref, a0_ref, a1_ref, bg_ref, wd_ref, bd_ref,
              out_ref):
    agg = a0_ref[0] + a1_ref[0]
    dinv = dinv_ref[...]
    nh = jnp.maximum(dinv * (agg + y_ref[...]) + bg_ref[...], 0.0)
    h2 = 0.5 * h_ref[...] + 0.5 * nh
    logits = _mm(h2, wd_ref[...]) + bd_ref[...]
    m = jnp.max(logits, axis=1, keepdims=True)
    lse = jnp.log(jnp.sum(jnp.exp(logits - m), axis=1, keepdims=True)) + m
    out_ref[...] = logits - lse


_fin = pl.pallas_call(
    _fin_body,
    grid=(GRID,),
    in_specs=[
        pl.BlockSpec((BROW, DH), _row),
        pl.BlockSpec((BROW, DH), _row),
        pl.BlockSpec((BROW, 1), _row),
        pl.BlockSpec((1, BROW, DH), _p0),
        pl.BlockSpec((1, BROW, DH), _p1),
        pl.BlockSpec((1, DH), _cst),
        pl.BlockSpec((DH, DO), _cst),
        pl.BlockSpec((1, DO), _cst),
    ],
    out_specs=pl.BlockSpec((BROW, DO), _row),
    out_shape=jax.ShapeDtypeStruct((N, DO), jnp.float32),
)


def kernel(x, edge_index, W_enc, b_enc, W_gc, b_gc, W_dec, b_dec):
    npt = EPT - E // NW
    ti = jnp.arange(NW, dtype=jnp.int32)[:, None]
    pj = jnp.arange(npt, dtype=jnp.int32)[None, :]
    pad_s = (pj * 89 + ti * 997) % N
    pad_d = N + (pj + ti * 7) % (NPAD - N)
    srcp = jnp.concatenate([edge_index[0].reshape(NW, -1), pad_s],
                           axis=1).reshape(NW * NCH, CHUNK)
    dstp = jnp.concatenate([edge_index[1].reshape(NW, -1), pad_d],
                           axis=1).reshape(NW * NCH, CHUNK)
    zer_r = jnp.zeros((CHUNK, DH), jnp.float32)
    be = b_enc.reshape(1, DH)
    bg = b_gc.reshape(1, DH)
    bd = b_dec.reshape(1, DO)

    degf = _deg(dstp).reshape(2, NPAD_D, 1)
    h, y, dinv = _enc(x, W_enc, be, W_gc, degf, degf)
    for _ in range(NUM_ITER - 1):
        aggf = _agg(y, srcp, dstp, zer_r).reshape(2, NPAD, DH)
        h, y = _upd(h, y, dinv, aggf, aggf, bg, W_gc)
    aggf = _agg(y, srcp, dstp, zer_r).reshape(2, NPAD, DH)
    return _fin(h, y, dinv, aggf, aggf, bg, W_dec, bd)
